# baseline (speedup 1.0000x reference)
import jax
import jax.numpy as jnp
from jax import lax
from jax.experimental import pallas as pl
from jax.experimental.pallas import tpu as pltpu
from jax.experimental.pallas import tpu_sc as plsc

D_MODEL = 4096
SEQ_LEN = 64
NUM_CORES = 1
NUM_SUBCORES = 16
NUM_WORKERS = NUM_CORES * NUM_SUBCORES
COLS = D_MODEL // NUM_WORKERS
LANES = 16
CCHUNKS = COLS // LANES


def _pe_body(abs_hbm, file_hbm, rank_hbm, diag_hbm, anti_hbm, out_hbm,
             acc, fb, rb, db, ab, sem):
    wid = lax.axis_index("s") * NUM_CORES + lax.axis_index("c")
    col0 = wid * COLS

    copies = [
        pltpu.async_copy(abs_hbm.at[:, pl.ds(col0, COLS)], acc, sem),
        pltpu.async_copy(file_hbm.at[:, pl.ds(col0, COLS)], fb, sem),
        pltpu.async_copy(rank_hbm.at[:, pl.ds(col0, COLS)], rb, sem),
        pltpu.async_copy(diag_hbm.at[:, pl.ds(col0, COLS)], db, sem),
        pltpu.async_copy(anti_hbm.at[:, pl.ds(col0, COLS)], ab, sem),
    ]
    for c in copies:
        c.wait()

    @plsc.parallel_loop(0, 8)
    def rank_body(k):
        k8 = k * 8
        rk = [rb[k, pl.ds(c * LANES, LANES)] for c in range(CCHUNKS)]
        for j in range(8):
            ts = []
            for c in range(CCHUNKS):
                off = c * LANES
                ts.append((fb[j, pl.ds(off, LANES)] + rk[c])
                          + (db[k + j, pl.ds(off, LANES)]
                             + ab[k - j + 7, pl.ds(off, LANES)]))
            for c in range(CCHUNKS):
                plsc.addupdate(acc.at[k8 + j, pl.ds(c * LANES, LANES)], ts[c])

    pltpu.sync_copy(acc, out_hbm.at[:, pl.ds(col0, COLS)])


@jax.jit
def _pos_encoding(abs_pos2d, file_table, rank_table, diag_table, anti_diag_table):
    run = pl.kernel(
        _pe_body,
        out_type=jax.ShapeDtypeStruct((SEQ_LEN, D_MODEL), jnp.float32),
        mesh=plsc.VectorSubcoreMesh(
            core_axis_name="c", subcore_axis_name="s",
            num_cores=NUM_CORES, num_subcores=NUM_SUBCORES),
        scratch_types=[
            pltpu.VMEM((SEQ_LEN, COLS), jnp.float32),
            pltpu.VMEM((8, COLS), jnp.float32),
            pltpu.VMEM((8, COLS), jnp.float32),
            pltpu.VMEM((15, COLS), jnp.float32),
            pltpu.VMEM((15, COLS), jnp.float32),
            pltpu.SemaphoreType.DMA,
        ],
    )
    return run(abs_pos2d, file_table, rank_table, diag_table, anti_diag_table)


def kernel(x, abs_pos, file_table, rank_table, diag_table, anti_diag_table):
    del x
    out = _pos_encoding(abs_pos.reshape(SEQ_LEN, D_MODEL),
                        file_table, rank_table, diag_table, anti_diag_table)
    return out.reshape(1, SEQ_LEN, D_MODEL)


# speedup vs baseline: 1.1075x; 1.1075x over previous
import jax
import jax.numpy as jnp
from jax import lax
from jax.experimental import pallas as pl
from jax.experimental.pallas import tpu as pltpu
from jax.experimental.pallas import tpu_sc as plsc

D_MODEL = 4096
SEQ_LEN = 64
NUM_CORES = 2
NUM_SUBCORES = 16
NUM_WORKERS = NUM_CORES * NUM_SUBCORES
COLS = D_MODEL // NUM_WORKERS
LANES = 16
CCHUNKS = COLS // LANES


def _pe_body(abs_hbm, file_hbm, rank_hbm, diag_hbm, anti_hbm, out_hbm,
             acc, fb, rb, db, ab, sem):
    wid = lax.axis_index("s") * NUM_CORES + lax.axis_index("c")
    col0 = wid * COLS

    copies = [
        pltpu.async_copy(abs_hbm.at[:, pl.ds(col0, COLS)], acc, sem),
        pltpu.async_copy(file_hbm.at[:, pl.ds(col0, COLS)], fb, sem),
        pltpu.async_copy(rank_hbm.at[:, pl.ds(col0, COLS)], rb, sem),
        pltpu.async_copy(diag_hbm.at[:, pl.ds(col0, COLS)], db, sem),
        pltpu.async_copy(anti_hbm.at[:, pl.ds(col0, COLS)], ab, sem),
    ]
    for c in copies:
        c.wait()

    @plsc.parallel_loop(0, SEQ_LEN, unroll=2)
    def row_body(i):
        k = lax.div(i, 8)
        f = lax.rem(i, 8)
        dg = k + f
        ad = k - f + 7
        ts = []
        for c in range(CCHUNKS):
            off = c * LANES
            ts.append((fb[f, pl.ds(off, LANES)] + rb[k, pl.ds(off, LANES)])
                      + (db[dg, pl.ds(off, LANES)] + ab[ad, pl.ds(off, LANES)]))
        for c in range(CCHUNKS):
            plsc.addupdate(acc.at[i, pl.ds(c * LANES, LANES)], ts[c])

    pltpu.sync_copy(acc, out_hbm.at[:, pl.ds(col0, COLS)])


@jax.jit
def _pos_encoding(abs_pos2d, file_table, rank_table, diag_table, anti_diag_table):
    run = pl.kernel(
        _pe_body,
        out_type=jax.ShapeDtypeStruct((SEQ_LEN, D_MODEL), jnp.float32),
        mesh=plsc.VectorSubcoreMesh(
            core_axis_name="c", subcore_axis_name="s",
            num_cores=NUM_CORES, num_subcores=NUM_SUBCORES),
        scratch_types=[
            pltpu.VMEM((SEQ_LEN, COLS), jnp.float32),
            pltpu.VMEM((8, COLS), jnp.float32),
            pltpu.VMEM((8, COLS), jnp.float32),
            pltpu.VMEM((15, COLS), jnp.float32),
            pltpu.VMEM((15, COLS), jnp.float32),
            pltpu.SemaphoreType.DMA,
        ],
    )
    return run(abs_pos2d, file_table, rank_table, diag_table, anti_diag_table)


def kernel(x, abs_pos, file_table, rank_table, diag_table, anti_diag_table):
    del x
    out = _pos_encoding(abs_pos.reshape(SEQ_LEN, D_MODEL),
                        file_table, rank_table, diag_table, anti_diag_table)
    return out.reshape(1, SEQ_LEN, D_MODEL)
